# Initial kernel scaffold; baseline (speedup 1.0000x reference)
import jax
import jax.numpy as jnp
from jax.experimental import pallas as pl


def kernel(node_feat, edge_feat, x_indices, mask_valid, ln_n_g, ln_n_b, W_node, b_node, ln_e_g, ln_e_b, W_edge, b_edge, W_skip, b_skip):
    raise NotImplementedError("write your pallas kernel here")


# speedup vs baseline: 2.6712x; 2.6712x over previous
import functools

import jax
import jax.numpy as jnp
from jax import lax
from jax.experimental import pallas as pl
from jax.experimental.pallas import tpu as pltpu
from jax.experimental.pallas import tpu_sc as plsc


def _nf_lin_body(nf_ref, g_ref, b_ref, w_ref, bias_ref, out_ref):
    x = nf_ref[...]
    mu = jnp.mean(x, axis=-1, keepdims=True)
    var = jnp.mean((x - mu) ** 2, axis=-1, keepdims=True)
    ln = (x - mu) * lax.rsqrt(var + 1e-5) * g_ref[...] + b_ref[...]
    out_ref[...] = (
        jnp.dot(ln, w_ref[...], preferred_element_type=jnp.float32) + bias_ref[...]
    )


def _nf_lin(nf_flat, g, b, w, bias, blk):
    rows, h = nf_flat.shape
    he = w.shape[1]
    grid = rows // blk
    return pl.pallas_call(
        _nf_lin_body,
        grid=(grid,),
        in_specs=[
            pl.BlockSpec((blk, h), lambda i: (i, 0)),
            pl.BlockSpec((1, h), lambda i: (0, 0)),
            pl.BlockSpec((1, h), lambda i: (0, 0)),
            pl.BlockSpec((h, he), lambda i: (0, 0)),
            pl.BlockSpec((1, he), lambda i: (0, 0)),
        ],
        out_specs=pl.BlockSpec((blk, he), lambda i: (i, 0)),
        out_shape=jax.ShapeDtypeStruct((rows, he), jnp.float32),
    )(nf_flat, g, b, w, bias)


def _make_gather_sum(Bn, En, Nn, Hn):
    info = plsc.get_sparse_core_info()
    NC, NS = info.num_cores, info.num_subcores
    NW = NC * NS
    R = Bn * En
    rows_w = R // NW
    CH = 128
    nch = rows_w // CH
    mesh = plsc.VectorSubcoreMesh(core_axis_name="c", subcore_axis_name="s")

    @functools.partial(
        pl.kernel,
        out_type=jax.ShapeDtypeStruct((R, Hn), jnp.float32),
        mesh=mesh,
        scratch_types=[
            pltpu.VMEM((nch, CH), jnp.int32),
            pltpu.VMEM((nch, CH), jnp.int32),
            pltpu.VMEM((CH, Hn), jnp.float32),
            pltpu.VMEM((CH, Hn), jnp.float32),
            pltpu.SemaphoreType.DMA,
            pltpu.SemaphoreType.DMA,
        ],
    )
    def gather_sum(nf_hbm, x0_hbm, x1_hbm, out_hbm, idx0, idx1, rows0, rows1, sem0, sem1):
        wid = lax.axis_index("s") * NC + lax.axis_index("c")
        base = wid * rows_w
        b = base // En
        bN = b * Nn
        r0 = pl.multiple_of((base % En) // CH, 8)
        pltpu.sync_copy(x0_hbm.at[pl.ds(r0, nch)], idx0)
        pltpu.sync_copy(x1_hbm.at[pl.ds(r0, nch)], idx1)

        off = jnp.full((16,), bN, jnp.int32)

        def add_off(i, _):
            r = i // (CH // 16)
            c = (i % (CH // 16)) * 16
            idx0[r, pl.ds(c, 16)] = idx0[r, pl.ds(c, 16)] + off
            idx1[r, pl.ds(c, 16)] = idx1[r, pl.ds(c, 16)] + off
            return 0

        lax.fori_loop(0, nch * (CH // 16), add_off, 0)

        def chunk(c, _):
            cp0 = pltpu.async_copy(nf_hbm.at[idx0.at[c]], rows0, sem0)
            cp1 = pltpu.async_copy(nf_hbm.at[idx1.at[c]], rows1, sem1)
            cp0.wait()
            cp1.wait()

            def add_rows(i, _):
                r = i // (Hn // 16)
                col = (i % (Hn // 16)) * 16
                rows0[r, pl.ds(col, 16)] = (
                    rows0[r, pl.ds(col, 16)] + rows1[r, pl.ds(col, 16)]
                )
                return 0

            lax.fori_loop(0, CH * (Hn // 16), add_rows, 0)
            pltpu.sync_copy(rows0, out_hbm.at[pl.ds(base + c * CH, CH)])
            return 0

        lax.fori_loop(0, nch, chunk, 0)

    return gather_sum


def _edge_mlp_body(ef_ref, ns_ref, m_ref, g_ref, b_ref, we_ref, be_ref, ws_ref,
                   bs_ref, out_ref):
    ef = ef_ref[...]
    mu = jnp.mean(ef, axis=-1, keepdims=True)
    var = jnp.mean((ef - mu) ** 2, axis=-1, keepdims=True)
    ln = (ef - mu) * lax.rsqrt(var + 1e-5) * g_ref[...] + b_ref[...]
    ef_lin = jnp.dot(ln, we_ref[...], preferred_element_type=jnp.float32) + be_ref[...]
    x = ef_lin + ns_ref[...]
    comb = 0.5 * x * (1.0 + lax.erf(x * 0.7071067811865476))
    out = ef + jnp.dot(comb, ws_ref[...], preferred_element_type=jnp.float32) + bs_ref[...]
    out_ref[...] = out * m_ref[...]


def _edge_mlp(ef_flat, ns_flat, m_flat, g, b, we, be, ws, bs, blk):
    rows, h = ef_flat.shape
    grid = rows // blk
    return pl.pallas_call(
        _edge_mlp_body,
        grid=(grid,),
        in_specs=[
            pl.BlockSpec((blk, h), lambda i: (i, 0)),
            pl.BlockSpec((blk, h), lambda i: (i, 0)),
            pl.BlockSpec((blk, 1), lambda i: (i, 0)),
            pl.BlockSpec((1, h), lambda i: (0, 0)),
            pl.BlockSpec((1, h), lambda i: (0, 0)),
            pl.BlockSpec((h, h), lambda i: (0, 0)),
            pl.BlockSpec((1, h), lambda i: (0, 0)),
            pl.BlockSpec((h, h), lambda i: (0, 0)),
            pl.BlockSpec((1, h), lambda i: (0, 0)),
        ],
        out_specs=pl.BlockSpec((blk, h), lambda i: (i, 0)),
        out_shape=jax.ShapeDtypeStruct((rows, h), jnp.float32),
    )(ef_flat, ns_flat, m_flat, g, b, we, be, ws, bs)


def kernel(node_feat, edge_feat, x_indices, mask_valid, ln_n_g, ln_n_b, W_node,
           b_node, ln_e_g, ln_e_b, W_edge, b_edge, W_skip, b_skip):
    Bn, Nn, Hn = node_feat.shape
    En = edge_feat.shape[1]
    He = edge_feat.shape[2]
    R = Bn * En
    CH = 128

    nf_flat = node_feat.reshape(Bn * Nn, Hn)
    nf_lin = _nf_lin(
        nf_flat,
        ln_n_g.reshape(1, Hn),
        ln_n_b.reshape(1, Hn),
        W_node,
        b_node.reshape(1, He),
        blk=1024,
    )

    x0 = x_indices[0].reshape(En // CH, CH)
    x1 = x_indices[1].reshape(En // CH, CH)
    node_sum = _make_gather_sum(Bn, En, Nn, He)(nf_lin, x0, x1)

    out_flat = _edge_mlp(
        edge_feat.reshape(R, He),
        node_sum,
        mask_valid.reshape(R, 1),
        ln_e_g.reshape(1, He),
        ln_e_b.reshape(1, He),
        W_edge,
        b_edge.reshape(1, He),
        W_skip,
        b_skip.reshape(1, He),
        blk=1024,
    )
    return out_flat.reshape(Bn, En, He)


# speedup vs baseline: 3.7901x; 1.4189x over previous
import functools

import jax
import jax.numpy as jnp
from jax import lax
from jax.experimental import pallas as pl
from jax.experimental.pallas import tpu as pltpu
from jax.experimental.pallas import tpu_sc as plsc


def _nf_lin_body(nf_ref, g_ref, b_ref, w_ref, bias_ref, out_ref):
    x = nf_ref[...]
    mu = jnp.mean(x, axis=-1, keepdims=True)
    var = jnp.mean((x - mu) ** 2, axis=-1, keepdims=True)
    ln = (x - mu) * lax.rsqrt(var + 1e-5) * g_ref[...] + b_ref[...]
    out_ref[...] = (
        jnp.dot(ln, w_ref[...], preferred_element_type=jnp.float32) + bias_ref[...]
    )


def _nf_lin(nf_flat, g, b, w, bias, blk):
    rows, h = nf_flat.shape
    he = w.shape[1]
    grid = rows // blk
    return pl.pallas_call(
        _nf_lin_body,
        grid=(grid,),
        in_specs=[
            pl.BlockSpec((blk, h), lambda i: (i, 0)),
            pl.BlockSpec((1, h), lambda i: (0, 0)),
            pl.BlockSpec((1, h), lambda i: (0, 0)),
            pl.BlockSpec((h, he), lambda i: (0, 0)),
            pl.BlockSpec((1, he), lambda i: (0, 0)),
        ],
        out_specs=pl.BlockSpec((blk, he), lambda i: (i, 0)),
        out_shape=jax.ShapeDtypeStruct((rows, he), jnp.float32),
    )(nf_flat, g, b, w, bias)


def _make_gather_sum(Bn, En, Nn, Hn):
    info = plsc.get_sparse_core_info()
    NC, NS = info.num_cores, info.num_subcores
    NW = NC * NS
    R = Bn * En
    rows_w = R // NW
    CH = 128
    nch = rows_w // CH
    mesh = plsc.VectorSubcoreMesh(core_axis_name="c", subcore_axis_name="s")

    assert nch % 2 == 0

    @functools.partial(
        pl.kernel,
        out_type=jax.ShapeDtypeStruct((R, Hn), jnp.float32),
        mesh=mesh,
        scratch_types=[
            pltpu.VMEM((nch, CH), jnp.int32),
            pltpu.VMEM((nch, CH), jnp.int32),
            pltpu.VMEM((CH, Hn), jnp.float32),
            pltpu.VMEM((CH, Hn), jnp.float32),
            pltpu.SemaphoreType.DMA,
            pltpu.SemaphoreType.DMA,
            pltpu.SemaphoreType.DMA,
            pltpu.SemaphoreType.DMA,
        ],
    )
    def gather_sum(nf_hbm, x0_hbm, x1_hbm, out_hbm, idx0, idx1, rbuf0, rbuf1,
                   sg0, sg1, st0, st1):
        wid = lax.axis_index("s") * NC + lax.axis_index("c")
        base = wid * rows_w
        b = base // En
        bN = b * Nn
        r0 = pl.multiple_of((base % En) // CH, 8)
        pltpu.sync_copy(x0_hbm.at[pl.ds(r0, nch)], idx0)
        pltpu.sync_copy(x1_hbm.at[pl.ds(r0, nch)], idx1)

        off = jnp.full((16,), bN, jnp.int32)

        def add_off(r, _):
            for buf in (idx0, idx1):
                for i in range(CH // 16):
                    buf[r, pl.ds(i * 16, 16)] = buf[r, pl.ds(i * 16, 16)] + off
            return 0

        lax.fori_loop(0, nch, add_off, 0)

        rbufs = (rbuf0, rbuf1)
        sgs = (sg0, sg1)
        sts = (st0, st1)

        def outer(c0, _):
            for s in range(2):
                c = c0 * 2 + s
                rb, sg, st = rbufs[s], sgs[s], sts[s]

                @pl.when(c0 > 0)
                def _():
                    pltpu.make_async_copy(rb, out_hbm.at[pl.ds(0, CH)], st).wait()

                pltpu.async_copy(nf_hbm.at[idx0.at[c]], rb, sg).wait()
                pltpu.async_copy(nf_hbm.at[idx1.at[c]], rb, sg, add=True).wait()
                dst = out_hbm.at[pl.ds(pl.multiple_of(base + c * CH, 8), CH)]
                pltpu.async_copy(rb, dst, st)
            return 0

        lax.fori_loop(0, nch // 2, outer, 0)
        for s in range(2):
            pltpu.make_async_copy(rbufs[s], out_hbm.at[pl.ds(0, CH)], sts[s]).wait()

    return gather_sum


def _edge_mlp_body(ef_ref, ns_ref, m_ref, g_ref, b_ref, we_ref, be_ref, ws_ref,
                   bs_ref, out_ref):
    ef = ef_ref[...]
    mu = jnp.mean(ef, axis=-1, keepdims=True)
    var = jnp.mean((ef - mu) ** 2, axis=-1, keepdims=True)
    ln = (ef - mu) * lax.rsqrt(var + 1e-5) * g_ref[...] + b_ref[...]
    ef_lin = jnp.dot(ln, we_ref[...], preferred_element_type=jnp.float32) + be_ref[...]
    x = ef_lin + ns_ref[...]
    comb = 0.5 * x * (1.0 + lax.erf(x * 0.7071067811865476))
    out = ef + jnp.dot(comb, ws_ref[...], preferred_element_type=jnp.float32) + bs_ref[...]
    out_ref[...] = out * m_ref[...]


def _edge_mlp(ef_flat, ns_flat, m_flat, g, b, we, be, ws, bs, blk):
    rows, h = ef_flat.shape
    grid = rows // blk
    return pl.pallas_call(
        _edge_mlp_body,
        grid=(grid,),
        in_specs=[
            pl.BlockSpec((blk, h), lambda i: (i, 0)),
            pl.BlockSpec((blk, h), lambda i: (i, 0)),
            pl.BlockSpec((blk, 1), lambda i: (i, 0)),
            pl.BlockSpec((1, h), lambda i: (0, 0)),
            pl.BlockSpec((1, h), lambda i: (0, 0)),
            pl.BlockSpec((h, h), lambda i: (0, 0)),
            pl.BlockSpec((1, h), lambda i: (0, 0)),
            pl.BlockSpec((h, h), lambda i: (0, 0)),
            pl.BlockSpec((1, h), lambda i: (0, 0)),
        ],
        out_specs=pl.BlockSpec((blk, h), lambda i: (i, 0)),
        out_shape=jax.ShapeDtypeStruct((rows, h), jnp.float32),
    )(ef_flat, ns_flat, m_flat, g, b, we, be, ws, bs)


def kernel(node_feat, edge_feat, x_indices, mask_valid, ln_n_g, ln_n_b, W_node,
           b_node, ln_e_g, ln_e_b, W_edge, b_edge, W_skip, b_skip):
    Bn, Nn, Hn = node_feat.shape
    En = edge_feat.shape[1]
    He = edge_feat.shape[2]
    R = Bn * En
    CH = 128

    nf_flat = node_feat.reshape(Bn * Nn, Hn)
    nf_lin = _nf_lin(
        nf_flat,
        ln_n_g.reshape(1, Hn),
        ln_n_b.reshape(1, Hn),
        W_node,
        b_node.reshape(1, He),
        blk=1024,
    )

    x0 = x_indices[0].reshape(En // CH, CH)
    x1 = x_indices[1].reshape(En // CH, CH)
    node_sum = _make_gather_sum(Bn, En, Nn, He)(nf_lin, x0, x1)

    out_flat = _edge_mlp(
        edge_feat.reshape(R, He),
        node_sum,
        mask_valid.reshape(R, 1),
        ln_e_g.reshape(1, He),
        ln_e_b.reshape(1, He),
        W_edge,
        b_edge.reshape(1, He),
        W_skip,
        b_skip.reshape(1, He),
        blk=1024,
    )
    return out_flat.reshape(Bn, En, He)


# speedup vs baseline: 4.7949x; 1.2651x over previous
import functools

import jax
import jax.numpy as jnp
from jax import lax
from jax.experimental import pallas as pl
from jax.experimental.pallas import tpu as pltpu
from jax.experimental.pallas import tpu_sc as plsc


def _nf_lin_body(nf_ref, g_ref, b_ref, w_ref, bias_ref, out_ref):
    x = nf_ref[...]
    mu = jnp.mean(x, axis=-1, keepdims=True)
    var = jnp.mean((x - mu) ** 2, axis=-1, keepdims=True)
    ln = (x - mu) * lax.rsqrt(var + 1e-5) * g_ref[...] + b_ref[...]
    out_ref[...] = (
        jnp.dot(ln, w_ref[...], preferred_element_type=jnp.float32) + bias_ref[...]
    )


def _nf_lin(nf_flat, g, b, w, bias, blk):
    rows, h = nf_flat.shape
    he = w.shape[1]
    grid = rows // blk
    return pl.pallas_call(
        _nf_lin_body,
        grid=(grid,),
        in_specs=[
            pl.BlockSpec((blk, h), lambda i: (i, 0)),
            pl.BlockSpec((1, h), lambda i: (0, 0)),
            pl.BlockSpec((1, h), lambda i: (0, 0)),
            pl.BlockSpec((h, he), lambda i: (0, 0)),
            pl.BlockSpec((1, he), lambda i: (0, 0)),
        ],
        out_specs=pl.BlockSpec((blk, he), lambda i: (i, 0)),
        out_shape=jax.ShapeDtypeStruct((rows, he), jnp.float32),
    )(nf_flat, g, b, w, bias)


def _make_gather_sum(Bn, En, Nn, Hn):
    info = plsc.get_sparse_core_info()
    NC, NS = info.num_cores, info.num_subcores
    NW = NC * NS
    R = Bn * En
    rows_w = R // NW
    CH = 128
    nch = rows_w // CH
    mesh = plsc.VectorSubcoreMesh(core_axis_name="c", subcore_axis_name="s")

    assert nch % 2 == 0
    rows_tile = (Bn * Nn) // NS

    @functools.partial(
        pl.kernel,
        out_type=jax.ShapeDtypeStruct((R, Hn), jnp.float32),
        mesh=mesh,
        scratch_types=[
            pltpu.VMEM_SHARED((Bn * Nn, Hn), jnp.float32),
            pltpu.VMEM((nch, CH), jnp.int32),
            pltpu.VMEM((nch, CH), jnp.int32),
            pltpu.VMEM((CH, Hn), jnp.float32),
            pltpu.VMEM((CH, Hn), jnp.float32),
            pltpu.SemaphoreType.DMA,
            pltpu.SemaphoreType.DMA,
            pltpu.SemaphoreType.DMA,
            pltpu.SemaphoreType.DMA,
        ],
    )
    def gather_sum(nf_hbm, x0_hbm, x1_hbm, out_hbm, table, idx0, idx1, rbuf0,
                   rbuf1, sg0, sg1, st0, st1):
        cid = lax.axis_index("c")
        sid = lax.axis_index("s")
        wid = sid * NC + cid
        base = wid * rows_w
        b = base // En
        bN = b * Nn
        toff = pl.multiple_of(sid * rows_tile, 8)
        stage = pltpu.async_copy(
            nf_hbm.at[pl.ds(toff, rows_tile)], table.at[pl.ds(toff, rows_tile)], st0
        )
        r0 = pl.multiple_of((base % En) // CH, 8)
        pltpu.sync_copy(x0_hbm.at[pl.ds(r0, nch)], idx0)
        pltpu.sync_copy(x1_hbm.at[pl.ds(r0, nch)], idx1)

        off = jnp.full((16,), bN, jnp.int32)

        def add_off(r, _):
            for buf in (idx0, idx1):
                for i in range(CH // 16):
                    buf[r, pl.ds(i * 16, 16)] = buf[r, pl.ds(i * 16, 16)] + off
            return 0

        lax.fori_loop(0, nch, add_off, 0)
        stage.wait()
        plsc.subcore_barrier()

        rbufs = (rbuf0, rbuf1)
        sgs = (sg0, sg1)
        sts = (st0, st1)

        def outer(c0, _):
            for s in range(2):
                c = c0 * 2 + s
                rb, sg, st = rbufs[s], sgs[s], sts[s]

                @pl.when(c0 > 0)
                def _():
                    pltpu.make_async_copy(rb, out_hbm.at[pl.ds(0, CH)], st).wait()

                pltpu.async_copy(table.at[idx0.at[c]], rb, sg).wait()
                pltpu.async_copy(table.at[idx1.at[c]], rb, sg, add=True).wait()
                dst = out_hbm.at[pl.ds(pl.multiple_of(base + c * CH, 8), CH)]
                pltpu.async_copy(rb, dst, st)
            return 0

        lax.fori_loop(0, nch // 2, outer, 0)
        for s in range(2):
            pltpu.make_async_copy(rbufs[s], out_hbm.at[pl.ds(0, CH)], sts[s]).wait()

    return gather_sum


def _edge_mlp_body(ef_ref, ns_ref, m_ref, g_ref, b_ref, we_ref, be_ref, ws_ref,
                   bs_ref, out_ref):
    ef = ef_ref[...]
    mu = jnp.mean(ef, axis=-1, keepdims=True)
    var = jnp.mean((ef - mu) ** 2, axis=-1, keepdims=True)
    ln = (ef - mu) * lax.rsqrt(var + 1e-5) * g_ref[...] + b_ref[...]
    ef_lin = jnp.dot(ln, we_ref[...], preferred_element_type=jnp.float32) + be_ref[...]
    x = ef_lin + ns_ref[...]
    comb = 0.5 * x * (1.0 + lax.erf(x * 0.7071067811865476))
    out = ef + jnp.dot(comb, ws_ref[...], preferred_element_type=jnp.float32) + bs_ref[...]
    out_ref[...] = out * m_ref[...]


def _edge_mlp(ef_flat, ns_flat, m_flat, g, b, we, be, ws, bs, blk):
    rows, h = ef_flat.shape
    grid = rows // blk
    return pl.pallas_call(
        _edge_mlp_body,
        grid=(grid,),
        in_specs=[
            pl.BlockSpec((blk, h), lambda i: (i, 0)),
            pl.BlockSpec((blk, h), lambda i: (i, 0)),
            pl.BlockSpec((blk, 1), lambda i: (i, 0)),
            pl.BlockSpec((1, h), lambda i: (0, 0)),
            pl.BlockSpec((1, h), lambda i: (0, 0)),
            pl.BlockSpec((h, h), lambda i: (0, 0)),
            pl.BlockSpec((1, h), lambda i: (0, 0)),
            pl.BlockSpec((h, h), lambda i: (0, 0)),
            pl.BlockSpec((1, h), lambda i: (0, 0)),
        ],
        out_specs=pl.BlockSpec((blk, h), lambda i: (i, 0)),
        out_shape=jax.ShapeDtypeStruct((rows, h), jnp.float32),
    )(ef_flat, ns_flat, m_flat, g, b, we, be, ws, bs)


def kernel(node_feat, edge_feat, x_indices, mask_valid, ln_n_g, ln_n_b, W_node,
           b_node, ln_e_g, ln_e_b, W_edge, b_edge, W_skip, b_skip):
    Bn, Nn, Hn = node_feat.shape
    En = edge_feat.shape[1]
    He = edge_feat.shape[2]
    R = Bn * En
    CH = 128

    nf_flat = node_feat.reshape(Bn * Nn, Hn)
    nf_lin = _nf_lin(
        nf_flat,
        ln_n_g.reshape(1, Hn),
        ln_n_b.reshape(1, Hn),
        W_node,
        b_node.reshape(1, He),
        blk=1024,
    )

    x0 = x_indices[0].reshape(En // CH, CH)
    x1 = x_indices[1].reshape(En // CH, CH)
    node_sum = _make_gather_sum(Bn, En, Nn, He)(nf_lin, x0, x1)

    out_flat = _edge_mlp(
        edge_feat.reshape(R, He),
        node_sum,
        mask_valid.reshape(R, 1),
        ln_e_g.reshape(1, He),
        ln_e_b.reshape(1, He),
        W_edge,
        b_edge.reshape(1, He),
        W_skip,
        b_skip.reshape(1, He),
        blk=1024,
    )
    return out_flat.reshape(Bn, En, He)


# speedup vs baseline: 5.8208x; 1.2140x over previous
import functools

import jax
import jax.numpy as jnp
from jax import lax
from jax.experimental import pallas as pl
from jax.experimental.pallas import tpu as pltpu
from jax.experimental.pallas import tpu_sc as plsc


def _nf_lin_body(nf_ref, g_ref, b_ref, w_ref, bias_ref, out_ref):
    x = nf_ref[...]
    mu = jnp.mean(x, axis=-1, keepdims=True)
    var = jnp.mean((x - mu) ** 2, axis=-1, keepdims=True)
    ln = (x - mu) * lax.rsqrt(var + 1e-5) * g_ref[...] + b_ref[...]
    out_ref[...] = (
        jnp.dot(ln, w_ref[...], preferred_element_type=jnp.float32) + bias_ref[...]
    )


def _nf_lin(nf_flat, g, b, w, bias, blk):
    rows, h = nf_flat.shape
    he = w.shape[1]
    grid = rows // blk
    return pl.pallas_call(
        _nf_lin_body,
        grid=(grid,),
        in_specs=[
            pl.BlockSpec((blk, h), lambda i: (i, 0)),
            pl.BlockSpec((1, h), lambda i: (0, 0)),
            pl.BlockSpec((1, h), lambda i: (0, 0)),
            pl.BlockSpec((h, he), lambda i: (0, 0)),
            pl.BlockSpec((1, he), lambda i: (0, 0)),
        ],
        out_specs=pl.BlockSpec((blk, he), lambda i: (i, 0)),
        out_shape=jax.ShapeDtypeStruct((rows, he), jnp.float32),
    )(nf_flat, g, b, w, bias)


def _make_gather_sum(Bn, En, Nn, Hn):
    info = plsc.get_sparse_core_info()
    NC, NS = info.num_cores, info.num_subcores
    NW = NC * NS
    R = Bn * En
    rows_w = R // NW
    CH = 128
    nch = rows_w // CH
    mesh = plsc.VectorSubcoreMesh(core_axis_name="c", subcore_axis_name="s")

    assert nch % 2 == 0
    rows_tile = (Bn * Nn) // NS

    @functools.partial(
        pl.kernel,
        out_type=jax.ShapeDtypeStruct((R, Hn), jnp.float32),
        mesh=mesh,
        scratch_types=[
            pltpu.VMEM_SHARED((Bn * Nn, Hn), jnp.float32),
            pltpu.VMEM((nch, CH), jnp.int32),
            pltpu.VMEM((nch, CH), jnp.int32),
            pltpu.VMEM((CH, Hn), jnp.float32),
            pltpu.VMEM((CH, Hn), jnp.float32),
            pltpu.SemaphoreType.DMA,
            pltpu.SemaphoreType.DMA,
            pltpu.SemaphoreType.DMA,
            pltpu.SemaphoreType.DMA,
        ],
    )
    def gather_sum(nf_hbm, x0_hbm, x1_hbm, out_hbm, table, idx0, idx1, rbuf0,
                   rbuf1, sg0, sg1, st0, st1):
        cid = lax.axis_index("c")
        sid = lax.axis_index("s")
        wid = sid * NC + cid
        base = wid * rows_w
        b = base // En
        bN = b * Nn
        toff = pl.multiple_of(sid * rows_tile, 8)
        stage = pltpu.async_copy(
            nf_hbm.at[pl.ds(toff, rows_tile)], table.at[pl.ds(toff, rows_tile)], st0
        )
        r0 = pl.multiple_of((base % En) // CH, 8)
        pltpu.sync_copy(x0_hbm.at[pl.ds(r0, nch)], idx0)
        pltpu.sync_copy(x1_hbm.at[pl.ds(r0, nch)], idx1)

        off = jnp.full((16,), bN, jnp.int32)

        def add_off(r, _):
            for buf in (idx0, idx1):
                for i in range(CH // 16):
                    buf[r, pl.ds(i * 16, 16)] = buf[r, pl.ds(i * 16, 16)] + off
            return 0

        lax.fori_loop(0, nch, add_off, 0)
        stage.wait()
        plsc.subcore_barrier()

        rbufs = (rbuf0, rbuf1)
        sgs = (sg0, sg1)
        sts = (st0, st1)

        def outer(c0, _):
            for s in range(2):
                c = c0 * 2 + s
                rb, sg, st = rbufs[s], sgs[s], sts[s]

                @pl.when(c0 > 0)
                def _():
                    pltpu.make_async_copy(rb, out_hbm.at[pl.ds(0, CH)], st).wait()

                pltpu.async_copy(table.at[idx0.at[c]], rb, sg).wait()
                pltpu.async_copy(table.at[idx1.at[c]], rb, sg, add=True).wait()
                dst = out_hbm.at[pl.ds(pl.multiple_of(base + c * CH, 8), CH)]
                pltpu.async_copy(rb, dst, st)
            return 0

        lax.fori_loop(0, nch // 2, outer, 0)
        for s in range(2):
            pltpu.make_async_copy(rbufs[s], out_hbm.at[pl.ds(0, CH)], sts[s]).wait()

    return gather_sum


def _edge_mlp_body(ef_ref, ns_ref, m_ref, we_ref, be_ref, ws_ref, bs_ref,
                   out_ref):
    ef = ef_ref[...]
    mu = jnp.mean(ef, axis=-1, keepdims=True)
    msq = jnp.mean(ef * ef, axis=-1, keepdims=True)
    norm = (ef - mu) * lax.rsqrt(msq - mu * mu + 1e-5)
    ef_lin = (
        jnp.dot(norm.astype(jnp.bfloat16), we_ref[...],
                preferred_element_type=jnp.float32) + be_ref[...]
    )
    x = ef_lin + ns_ref[...]
    comb = 0.5 * x * (1.0 + lax.erf(x * 0.7071067811865476))
    out = ef + (
        jnp.dot(comb.astype(jnp.bfloat16), ws_ref[...],
                preferred_element_type=jnp.float32) + bs_ref[...]
    )
    out_ref[...] = out * m_ref[...]


def _edge_mlp(ef_flat, ns_flat, m_flat, we, be, ws, bs, blk):
    rows, h = ef_flat.shape
    grid = rows // blk
    return pl.pallas_call(
        _edge_mlp_body,
        grid=(grid,),
        in_specs=[
            pl.BlockSpec((blk, h), lambda i: (i, 0)),
            pl.BlockSpec((blk, h), lambda i: (i, 0)),
            pl.BlockSpec((blk, 1), lambda i: (i, 0)),
            pl.BlockSpec((h, h), lambda i: (0, 0)),
            pl.BlockSpec((1, h), lambda i: (0, 0)),
            pl.BlockSpec((h, h), lambda i: (0, 0)),
            pl.BlockSpec((1, h), lambda i: (0, 0)),
        ],
        out_specs=pl.BlockSpec((blk, h), lambda i: (i, 0)),
        out_shape=jax.ShapeDtypeStruct((rows, h), jnp.float32),
    )(ef_flat, ns_flat, m_flat, we, be, ws, bs)


def kernel(node_feat, edge_feat, x_indices, mask_valid, ln_n_g, ln_n_b, W_node,
           b_node, ln_e_g, ln_e_b, W_edge, b_edge, W_skip, b_skip):
    Bn, Nn, Hn = node_feat.shape
    En = edge_feat.shape[1]
    He = edge_feat.shape[2]
    R = Bn * En
    CH = 128

    nf_flat = node_feat.reshape(Bn * Nn, Hn)
    nf_lin = _nf_lin(
        nf_flat,
        ln_n_g.reshape(1, Hn),
        ln_n_b.reshape(1, Hn),
        W_node,
        b_node.reshape(1, He),
        blk=1024,
    )

    x0 = x_indices[0].reshape(En // CH, CH)
    x1 = x_indices[1].reshape(En // CH, CH)
    node_sum = _make_gather_sum(Bn, En, Nn, He)(nf_lin, x0, x1)

    we_eff = (ln_e_g[:, None] * W_edge).astype(jnp.bfloat16)
    be_eff = (ln_e_b @ W_edge + b_edge).reshape(1, He)
    out_flat = _edge_mlp(
        edge_feat.reshape(R, He),
        node_sum,
        mask_valid.reshape(R, 1),
        we_eff,
        be_eff,
        W_skip.astype(jnp.bfloat16),
        b_skip.reshape(1, He),
        blk=2048,
    )
    return out_flat.reshape(Bn, En, He)


# speedup vs baseline: 6.0909x; 1.0464x over previous
import functools

import jax
import jax.numpy as jnp
from jax import lax
from jax.experimental import pallas as pl
from jax.experimental.pallas import tpu as pltpu
from jax.experimental.pallas import tpu_sc as plsc


def _nf_lin_body(nf_ref, g_ref, b_ref, w_ref, bias_ref, out_ref):
    x = nf_ref[...]
    mu = jnp.mean(x, axis=-1, keepdims=True)
    var = jnp.mean((x - mu) ** 2, axis=-1, keepdims=True)
    ln = (x - mu) * lax.rsqrt(var + 1e-5) * g_ref[...] + b_ref[...]
    out_ref[...] = (
        jnp.dot(ln, w_ref[...], preferred_element_type=jnp.float32) + bias_ref[...]
    )


def _nf_lin(nf_flat, g, b, w, bias, blk):
    rows, h = nf_flat.shape
    he = w.shape[1]
    grid = rows // blk
    return pl.pallas_call(
        _nf_lin_body,
        grid=(grid,),
        in_specs=[
            pl.BlockSpec((blk, h), lambda i: (i, 0)),
            pl.BlockSpec((1, h), lambda i: (0, 0)),
            pl.BlockSpec((1, h), lambda i: (0, 0)),
            pl.BlockSpec((h, he), lambda i: (0, 0)),
            pl.BlockSpec((1, he), lambda i: (0, 0)),
        ],
        out_specs=pl.BlockSpec((blk, he), lambda i: (i, 0)),
        out_shape=jax.ShapeDtypeStruct((rows, he), jnp.float32),
    )(nf_flat, g, b, w, bias)


def _make_gather_sum_slice(b_idx, En, Nn, Hn):
    info = plsc.get_sparse_core_info()
    NC, NS = info.num_cores, info.num_subcores
    NW = NC * NS
    rows_w = En // NW
    CH = 128
    nch = rows_w // CH
    assert nch % 2 == 0
    rows_tile = Nn // NS
    mesh = plsc.VectorSubcoreMesh(core_axis_name="c", subcore_axis_name="s")

    @functools.partial(
        pl.kernel,
        out_type=jax.ShapeDtypeStruct((En, Hn), jnp.float32),
        mesh=mesh,
        scratch_types=[
            pltpu.VMEM_SHARED((Nn, Hn), jnp.float32),
            pltpu.VMEM((nch, CH), jnp.int32),
            pltpu.VMEM((nch, CH), jnp.int32),
            pltpu.VMEM((CH, Hn), jnp.float32),
            pltpu.VMEM((CH, Hn), jnp.float32),
            pltpu.SemaphoreType.DMA,
            pltpu.SemaphoreType.DMA,
            pltpu.SemaphoreType.DMA,
            pltpu.SemaphoreType.DMA,
        ],
    )
    def gather_sum(nf_hbm, x0_hbm, x1_hbm, out_hbm, table, idx0, idx1, rbuf0,
                   rbuf1, sg0, sg1, st0, st1):
        cid = lax.axis_index("c")
        sid = lax.axis_index("s")
        wid = sid * NC + cid
        base = wid * rows_w
        toff = pl.multiple_of(sid * rows_tile, 8)
        stage = pltpu.async_copy(
            nf_hbm.at[pl.ds(b_idx * Nn + toff, rows_tile)],
            table.at[pl.ds(toff, rows_tile)],
            st0,
        )
        r0 = pl.multiple_of(base // CH, 8)
        pltpu.sync_copy(x0_hbm.at[pl.ds(r0, nch)], idx0)
        pltpu.sync_copy(x1_hbm.at[pl.ds(r0, nch)], idx1)
        stage.wait()
        plsc.subcore_barrier()

        rbufs = (rbuf0, rbuf1)
        sgs = (sg0, sg1)
        sts = (st0, st1)

        def outer(c0, _):
            for s in range(2):
                c = c0 * 2 + s
                rb, sg, st = rbufs[s], sgs[s], sts[s]

                @pl.when(c0 > 0)
                def _():
                    pltpu.make_async_copy(rb, out_hbm.at[pl.ds(0, CH)], st).wait()

                pltpu.async_copy(table.at[idx0.at[c]], rb, sg).wait()
                pltpu.async_copy(table.at[idx1.at[c]], rb, sg, add=True).wait()
                dst = out_hbm.at[pl.ds(pl.multiple_of(base + c * CH, 8), CH)]
                pltpu.async_copy(rb, dst, st)
            return 0

        lax.fori_loop(0, nch // 2, outer, 0)
        for s in range(2):
            pltpu.make_async_copy(rbufs[s], out_hbm.at[pl.ds(0, CH)], sts[s]).wait()

    return gather_sum


def _edge_mlp_body(ef_ref, ns_ref, m_ref, we_ref, be_ref, ws_ref, bs_ref,
                   out_ref):
    ef = ef_ref[...]
    mu = jnp.mean(ef, axis=-1, keepdims=True)
    msq = jnp.mean(ef * ef, axis=-1, keepdims=True)
    norm = (ef - mu) * lax.rsqrt(msq - mu * mu + 1e-5)
    ef_lin = (
        jnp.dot(norm.astype(jnp.bfloat16), we_ref[...],
                preferred_element_type=jnp.float32) + be_ref[...]
    )
    x = ef_lin + ns_ref[...]
    comb = 0.5 * x * (1.0 + lax.erf(x * 0.7071067811865476))
    out = ef + (
        jnp.dot(comb.astype(jnp.bfloat16), ws_ref[...],
                preferred_element_type=jnp.float32) + bs_ref[...]
    )
    out_ref[...] = out * m_ref[...]


def _edge_mlp_slice_first(ef_flat, ns_b, m_flat, we, be, ws, bs, blk):
    rows, h = ef_flat.shape
    en = ns_b.shape[0]
    grid = en // blk
    return pl.pallas_call(
        _edge_mlp_body,
        grid=(grid,),
        in_specs=[
            pl.BlockSpec((blk, h), lambda i: (i, 0)),
            pl.BlockSpec((blk, h), lambda i: (i, 0)),
            pl.BlockSpec((blk, 1), lambda i: (i, 0)),
            pl.BlockSpec((h, h), lambda i: (0, 0)),
            pl.BlockSpec((1, h), lambda i: (0, 0)),
            pl.BlockSpec((h, h), lambda i: (0, 0)),
            pl.BlockSpec((1, h), lambda i: (0, 0)),
        ],
        out_specs=pl.BlockSpec((blk, h), lambda i: (i, 0)),
        out_shape=jax.ShapeDtypeStruct((rows, h), jnp.float32),
    )(ef_flat, ns_b, m_flat, we, be, ws, bs)


def _edge_mlp_body_acc(ef_ref, ns_ref, m_ref, we_ref, be_ref, ws_ref, bs_ref,
                       prev_ref, out_ref):
    del prev_ref
    _edge_mlp_body(ef_ref, ns_ref, m_ref, we_ref, be_ref, ws_ref, bs_ref,
                   out_ref)


def _edge_mlp_slice_acc(k0, ef_flat, ns_b, m_flat, we, be, ws, bs, prev, blk):
    rows, h = ef_flat.shape
    en = ns_b.shape[0]
    grid = en // blk
    return pl.pallas_call(
        _edge_mlp_body_acc,
        grid=(grid,),
        in_specs=[
            pl.BlockSpec((blk, h), lambda i: (k0 + i, 0)),
            pl.BlockSpec((blk, h), lambda i: (i, 0)),
            pl.BlockSpec((blk, 1), lambda i: (k0 + i, 0)),
            pl.BlockSpec((h, h), lambda i: (0, 0)),
            pl.BlockSpec((1, h), lambda i: (0, 0)),
            pl.BlockSpec((h, h), lambda i: (0, 0)),
            pl.BlockSpec((1, h), lambda i: (0, 0)),
            pl.BlockSpec((8, h), lambda i: (0, 0)),
        ],
        out_specs=pl.BlockSpec((blk, h), lambda i: (k0 + i, 0)),
        out_shape=jax.ShapeDtypeStruct((rows, h), jnp.float32),
        input_output_aliases={7: 0},
    )(ef_flat, ns_b, m_flat, we, be, ws, bs, prev)


def kernel(node_feat, edge_feat, x_indices, mask_valid, ln_n_g, ln_n_b, W_node,
           b_node, ln_e_g, ln_e_b, W_edge, b_edge, W_skip, b_skip):
    Bn, Nn, Hn = node_feat.shape
    En = edge_feat.shape[1]
    He = edge_feat.shape[2]
    R = Bn * En
    CH = 128
    BLK = 2048

    nf_flat = node_feat.reshape(Bn * Nn, Hn)
    nf_lin = _nf_lin(
        nf_flat,
        ln_n_g.reshape(1, Hn),
        ln_n_b.reshape(1, Hn),
        W_node,
        b_node.reshape(1, He),
        blk=1024,
    )

    x0 = x_indices[0].reshape(En // CH, CH)
    x1 = x_indices[1].reshape(En // CH, CH)
    node_sums = [
        _make_gather_sum_slice(b, En, Nn, He)(nf_lin, x0, x1) for b in range(Bn)
    ]

    we_eff = (ln_e_g[:, None] * W_edge).astype(jnp.bfloat16)
    be_eff = (ln_e_b @ W_edge + b_edge).reshape(1, He)
    ws_bf = W_skip.astype(jnp.bfloat16)
    bs2 = b_skip.reshape(1, He)
    ef_flat = edge_feat.reshape(R, He)
    m_flat = mask_valid.reshape(R, 1)

    out = _edge_mlp_slice_first(ef_flat, node_sums[0], m_flat, we_eff, be_eff,
                                ws_bf, bs2, blk=BLK)
    for b in range(1, Bn):
        out = _edge_mlp_slice_acc(b * (En // BLK), ef_flat, node_sums[b],
                                  m_flat, we_eff, be_eff, ws_bf, bs2, out,
                                  blk=BLK)
    return out.reshape(Bn, En, He)


# speedup vs baseline: 6.6649x; 1.0942x over previous
import functools

import jax
import jax.numpy as jnp
from jax import lax
from jax.experimental import pallas as pl
from jax.experimental.pallas import tpu as pltpu
from jax.experimental.pallas import tpu_sc as plsc


def _nf_lin_body(nf_ref, g_ref, b_ref, w_ref, bias_ref, out_ref):
    x = nf_ref[...]
    mu = jnp.mean(x, axis=-1, keepdims=True)
    var = jnp.mean((x - mu) ** 2, axis=-1, keepdims=True)
    ln = (x - mu) * lax.rsqrt(var + 1e-5) * g_ref[...] + b_ref[...]
    out_ref[...] = (
        jnp.dot(ln, w_ref[...], preferred_element_type=jnp.float32) + bias_ref[...]
    )


def _nf_lin(nf_flat, g, b, w, bias, blk):
    rows, h = nf_flat.shape
    he = w.shape[1]
    grid = rows // blk
    return pl.pallas_call(
        _nf_lin_body,
        grid=(grid,),
        in_specs=[
            pl.BlockSpec((blk, h), lambda i: (i, 0)),
            pl.BlockSpec((1, h), lambda i: (0, 0)),
            pl.BlockSpec((1, h), lambda i: (0, 0)),
            pl.BlockSpec((h, he), lambda i: (0, 0)),
            pl.BlockSpec((1, he), lambda i: (0, 0)),
        ],
        out_specs=pl.BlockSpec((blk, he), lambda i: (i, 0)),
        out_shape=jax.ShapeDtypeStruct((rows, he), jnp.float32),
    )(nf_flat, g, b, w, bias)


def _make_gather_sum_slice(b_idx, En, Nn, Hn):
    info = plsc.get_sparse_core_info()
    NC, NS = info.num_cores, info.num_subcores
    NW = NC * NS
    rows_w = En // NW
    CH = 128
    nch = rows_w // CH
    assert nch % 2 == 0
    rows_tile = Nn // NS
    mesh = plsc.VectorSubcoreMesh(core_axis_name="c", subcore_axis_name="s")

    @functools.partial(
        pl.kernel,
        out_type=jax.ShapeDtypeStruct((En, Hn), jnp.float32),
        mesh=mesh,
        scratch_types=[
            pltpu.VMEM_SHARED((Nn, Hn), jnp.float32),
            pltpu.VMEM((nch, CH), jnp.int32),
            pltpu.VMEM((nch, CH), jnp.int32),
            pltpu.VMEM((CH, Hn), jnp.float32),
            pltpu.VMEM((CH, Hn), jnp.float32),
            pltpu.SemaphoreType.DMA,
            pltpu.SemaphoreType.DMA,
            pltpu.SemaphoreType.DMA,
            pltpu.SemaphoreType.DMA,
        ],
    )
    def gather_sum(nf_hbm, x0_hbm, x1_hbm, out_hbm, table, idx0, idx1, rbuf0,
                   rbuf1, sg0, sg1, st0, st1):
        cid = lax.axis_index("c")
        sid = lax.axis_index("s")
        wid = sid * NC + cid
        base = wid * rows_w
        toff = pl.multiple_of(sid * rows_tile, 8)
        stage = pltpu.async_copy(
            nf_hbm.at[pl.ds(b_idx * Nn + toff, rows_tile)],
            table.at[pl.ds(toff, rows_tile)],
            st0,
        )
        r0 = pl.multiple_of(base // CH, 8)
        pltpu.sync_copy(x0_hbm.at[pl.ds(r0, nch)], idx0)
        pltpu.sync_copy(x1_hbm.at[pl.ds(r0, nch)], idx1)
        stage.wait()
        plsc.subcore_barrier()

        rbufs = (rbuf0, rbuf1)
        sgs = (sg0, sg1)
        sts = (st0, st1)

        def outer(c0, _):
            for s in range(2):
                c = c0 * 2 + s
                rb, sg, st = rbufs[s], sgs[s], sts[s]

                @pl.when(c0 > 0)
                def _():
                    pltpu.make_async_copy(rb, out_hbm.at[pl.ds(0, CH)], st).wait()

                pltpu.async_copy(table.at[idx0.at[c]], rb, sg).wait()
                pltpu.async_copy(table.at[idx1.at[c]], rb, sg, add=True).wait()
                dst = out_hbm.at[pl.ds(pl.multiple_of(base + c * CH, 8), CH)]
                pltpu.async_copy(rb, dst, st)
            return 0

        lax.fori_loop(0, nch // 2, outer, 0)
        for s in range(2):
            pltpu.make_async_copy(rbufs[s], out_hbm.at[pl.ds(0, CH)], sts[s]).wait()

    return gather_sum


def _edge_mlp_body(ef_ref, ns_ref, m_ref, we_ref, be_ref, ws_ref, bs_ref,
                   out_ref):
    ef = ef_ref[...]
    mu = jnp.mean(ef, axis=-1, keepdims=True)
    msq = jnp.mean(ef * ef, axis=-1, keepdims=True)
    norm = (ef - mu) * lax.rsqrt(msq - mu * mu + 1e-5)
    ef_lin = (
        jnp.dot(norm.astype(jnp.bfloat16), we_ref[...],
                preferred_element_type=jnp.float32) + be_ref[...]
    )
    x = ef_lin + ns_ref[...]
    comb = 0.5 * x * (1.0 + lax.erf(x * 0.7071067811865476))
    out = ef + (
        jnp.dot(comb.astype(jnp.bfloat16), ws_ref[...],
                preferred_element_type=jnp.float32) + bs_ref[...]
    )
    out_ref[...] = out * m_ref[...]


def _edge_mlp_slice_first(ef_flat, ns_b, m_flat, we, be, ws, bs, blk):
    rows, h = ef_flat.shape
    en = ns_b.shape[0]
    grid = en // blk
    return pl.pallas_call(
        _edge_mlp_body,
        grid=(grid,),
        in_specs=[
            pl.BlockSpec((blk, h), lambda i: (i, 0)),
            pl.BlockSpec((blk, h), lambda i: (i, 0)),
            pl.BlockSpec((blk, 1), lambda i: (i, 0)),
            pl.BlockSpec((h, h), lambda i: (0, 0)),
            pl.BlockSpec((1, h), lambda i: (0, 0)),
            pl.BlockSpec((h, h), lambda i: (0, 0)),
            pl.BlockSpec((1, h), lambda i: (0, 0)),
        ],
        out_specs=pl.BlockSpec((blk, h), lambda i: (i, 0)),
        out_shape=jax.ShapeDtypeStruct((rows, h), jnp.float32),
    )(ef_flat, ns_b, m_flat, we, be, ws, bs)


def _edge_mlp_body_acc(ef_ref, ns_ref, m_ref, we_ref, be_ref, ws_ref, bs_ref,
                       prev_ref, out_ref):
    del prev_ref
    _edge_mlp_body(ef_ref, ns_ref, m_ref, we_ref, be_ref, ws_ref, bs_ref,
                   out_ref)


def _edge_mlp_slice_acc(k0, ef_flat, ns_b, m_flat, we, be, ws, bs, prev, blk):
    rows, h = ef_flat.shape
    en = ns_b.shape[0]
    grid = en // blk
    return pl.pallas_call(
        _edge_mlp_body_acc,
        grid=(grid,),
        in_specs=[
            pl.BlockSpec((blk, h), lambda i: (k0 + i, 0)),
            pl.BlockSpec((blk, h), lambda i: (i, 0)),
            pl.BlockSpec((blk, 1), lambda i: (k0 + i, 0)),
            pl.BlockSpec((h, h), lambda i: (0, 0)),
            pl.BlockSpec((1, h), lambda i: (0, 0)),
            pl.BlockSpec((h, h), lambda i: (0, 0)),
            pl.BlockSpec((1, h), lambda i: (0, 0)),
            pl.BlockSpec((8, h), lambda i: (0, 0)),
        ],
        out_specs=pl.BlockSpec((blk, h), lambda i: (k0 + i, 0)),
        out_shape=jax.ShapeDtypeStruct((rows, h), jnp.float32),
        input_output_aliases={7: 0},
    )(ef_flat, ns_b, m_flat, we, be, ws, bs, prev)


def kernel(node_feat, edge_feat, x_indices, mask_valid, ln_n_g, ln_n_b, W_node,
           b_node, ln_e_g, ln_e_b, W_edge, b_edge, W_skip, b_skip):
    Bn, Nn, Hn = node_feat.shape
    En = edge_feat.shape[1]
    He = edge_feat.shape[2]
    R = Bn * En
    CH = 128
    BLK = 4096

    nf_flat = node_feat.reshape(Bn * Nn, Hn)
    nf_lin = _nf_lin(
        nf_flat,
        ln_n_g.reshape(1, Hn),
        ln_n_b.reshape(1, Hn),
        W_node,
        b_node.reshape(1, He),
        blk=1024,
    )

    x0 = x_indices[0].reshape(En // CH, CH)
    x1 = x_indices[1].reshape(En // CH, CH)
    node_sums = [
        _make_gather_sum_slice(b, En, Nn, He)(nf_lin, x0, x1) for b in range(Bn)
    ]

    we_eff = (ln_e_g[:, None] * W_edge).astype(jnp.bfloat16)
    be_eff = (ln_e_b @ W_edge + b_edge).reshape(1, He)
    ws_bf = W_skip.astype(jnp.bfloat16)
    bs2 = b_skip.reshape(1, He)
    ef_flat = edge_feat.reshape(R, He)
    m_flat = mask_valid.reshape(R, 1)

    out = _edge_mlp_slice_first(ef_flat, node_sums[0], m_flat, we_eff, be_eff,
                                ws_bf, bs2, blk=BLK)
    for b in range(1, Bn):
        out = _edge_mlp_slice_acc(b * (En // BLK), ef_flat, node_sums[b],
                                  m_flat, we_eff, be_eff, ws_bf, bs2, out,
                                  blk=BLK)
    return out.reshape(Bn, En, He)


# speedup vs baseline: 6.7917x; 1.0190x over previous
import functools

import jax
import jax.numpy as jnp
from jax import lax
from jax.experimental import pallas as pl
from jax.experimental.pallas import tpu as pltpu
from jax.experimental.pallas import tpu_sc as plsc


def _nf_lin_body(nf_ref, g_ref, b_ref, w_ref, bias_ref, out_ref):
    x = nf_ref[...]
    mu = jnp.mean(x, axis=-1, keepdims=True)
    var = jnp.mean((x - mu) ** 2, axis=-1, keepdims=True)
    ln = (x - mu) * lax.rsqrt(var + 1e-5) * g_ref[...] + b_ref[...]
    out_ref[...] = (
        jnp.dot(ln, w_ref[...], preferred_element_type=jnp.float32) + bias_ref[...]
    )


def _nf_lin(nf_flat, g, b, w, bias, blk):
    rows, h = nf_flat.shape
    he = w.shape[1]
    grid = rows // blk
    return pl.pallas_call(
        _nf_lin_body,
        grid=(grid,),
        in_specs=[
            pl.BlockSpec((blk, h), lambda i: (i, 0)),
            pl.BlockSpec((1, h), lambda i: (0, 0)),
            pl.BlockSpec((1, h), lambda i: (0, 0)),
            pl.BlockSpec((h, he), lambda i: (0, 0)),
            pl.BlockSpec((1, he), lambda i: (0, 0)),
        ],
        out_specs=pl.BlockSpec((blk, he), lambda i: (i, 0)),
        out_shape=jax.ShapeDtypeStruct((rows, he), jnp.float32),
    )(nf_flat, g, b, w, bias)


def _make_gather_sum_slice(b_idx, En, Nn, Hn):
    info = plsc.get_sparse_core_info()
    NC, NS = info.num_cores, info.num_subcores
    NW = NC * NS
    rows_w = En // NW
    CH = 128
    nch = rows_w // CH
    assert nch % 2 == 0
    rows_tile = Nn // NS
    mesh = plsc.VectorSubcoreMesh(core_axis_name="c", subcore_axis_name="s")

    @functools.partial(
        pl.kernel,
        out_type=jax.ShapeDtypeStruct((En, Hn), jnp.float32),
        mesh=mesh,
        scratch_types=[
            pltpu.VMEM_SHARED((Nn, Hn), jnp.float32),
            pltpu.VMEM((nch, CH), jnp.int32),
            pltpu.VMEM((nch, CH), jnp.int32),
            pltpu.VMEM((CH, Hn), jnp.float32),
            pltpu.VMEM((CH, Hn), jnp.float32),
            pltpu.SemaphoreType.DMA,
            pltpu.SemaphoreType.DMA,
            pltpu.SemaphoreType.DMA,
            pltpu.SemaphoreType.DMA,
        ],
    )
    def gather_sum(nf_hbm, x0_hbm, x1_hbm, out_hbm, table, idx0, idx1, rbuf0,
                   rbuf1, sg0, sg1, st0, st1):
        cid = lax.axis_index("c")
        sid = lax.axis_index("s")
        wid = sid * NC + cid
        base = wid * rows_w
        toff = pl.multiple_of(sid * rows_tile, 8)
        stage = pltpu.async_copy(
            nf_hbm.at[pl.ds(b_idx * Nn + toff, rows_tile)],
            table.at[pl.ds(toff, rows_tile)],
            st0,
        )
        r0 = pl.multiple_of(base // CH, 8)
        pltpu.sync_copy(x0_hbm.at[pl.ds(r0, nch)], idx0)
        pltpu.sync_copy(x1_hbm.at[pl.ds(r0, nch)], idx1)
        stage.wait()
        plsc.subcore_barrier()

        rbufs = (rbuf0, rbuf1)
        sgs = (sg0, sg1)
        sts = (st0, st1)

        def outer(c0, _):
            for s in range(2):
                c = c0 * 2 + s
                rb, sg, st = rbufs[s], sgs[s], sts[s]

                @pl.when(c0 > 0)
                def _():
                    pltpu.make_async_copy(rb, out_hbm.at[pl.ds(0, CH)], st).wait()

                pltpu.async_copy(table.at[idx0.at[c]], rb, sg).wait()
                pltpu.async_copy(table.at[idx1.at[c]], rb, sg, add=True).wait()
                dst = out_hbm.at[pl.ds(pl.multiple_of(base + c * CH, 8), CH)]
                pltpu.async_copy(rb, dst, st)
            return 0

        lax.fori_loop(0, nch // 2, outer, 0)
        for s in range(2):
            pltpu.make_async_copy(rbufs[s], out_hbm.at[pl.ds(0, CH)], sts[s]).wait()

    return gather_sum


def _edge_mlp_body(ef_ref, ns_ref, m_ref, we_ref, be_ref, ws_ref, bs_ref,
                   out_ref):
    ef = ef_ref[...]
    mu = jnp.mean(ef, axis=-1, keepdims=True)
    msq = jnp.mean(ef * ef, axis=-1, keepdims=True)
    norm = (ef - mu) * lax.rsqrt(msq - mu * mu + 1e-5)
    ef_lin = (
        jnp.dot(norm.astype(jnp.bfloat16), we_ref[...],
                preferred_element_type=jnp.float32) + be_ref[...]
    )
    x = ef_lin + ns_ref[...]
    comb = 0.5 * x * (1.0 + lax.erf(x * 0.7071067811865476))
    out = ef + (
        jnp.dot(comb.astype(jnp.bfloat16), ws_ref[...],
                preferred_element_type=jnp.float32) + bs_ref[...]
    )
    out_ref[...] = out * m_ref[...]


def _edge_mlp_slice_first(ef_flat, ns_b, m_flat, we, be, ws, bs, blk):
    rows, h = ef_flat.shape
    en = ns_b.shape[0]
    grid = en // blk
    return pl.pallas_call(
        _edge_mlp_body,
        grid=(grid,),
        in_specs=[
            pl.BlockSpec((blk, h), lambda i: (i, 0)),
            pl.BlockSpec((blk, h), lambda i: (i, 0)),
            pl.BlockSpec((blk, 1), lambda i: (i, 0)),
            pl.BlockSpec((h, h), lambda i: (0, 0)),
            pl.BlockSpec((1, h), lambda i: (0, 0)),
            pl.BlockSpec((h, h), lambda i: (0, 0)),
            pl.BlockSpec((1, h), lambda i: (0, 0)),
        ],
        out_specs=pl.BlockSpec((blk, h), lambda i: (i, 0)),
        out_shape=jax.ShapeDtypeStruct((rows, h), jnp.float32),
    )(ef_flat, ns_b, m_flat, we, be, ws, bs)


def _edge_mlp_body_acc(ef_ref, ns_ref, m_ref, we_ref, be_ref, ws_ref, bs_ref,
                       prev_ref, out_ref):
    del prev_ref
    _edge_mlp_body(ef_ref, ns_ref, m_ref, we_ref, be_ref, ws_ref, bs_ref,
                   out_ref)


def _edge_mlp_slice_acc(k0, ef_flat, ns_b, m_flat, we, be, ws, bs, prev, blk):
    rows, h = ef_flat.shape
    en = ns_b.shape[0]
    grid = en // blk
    return pl.pallas_call(
        _edge_mlp_body_acc,
        grid=(grid,),
        in_specs=[
            pl.BlockSpec((blk, h), lambda i: (k0 + i, 0)),
            pl.BlockSpec((blk, h), lambda i: (i, 0)),
            pl.BlockSpec((blk, 1), lambda i: (k0 + i, 0)),
            pl.BlockSpec((h, h), lambda i: (0, 0)),
            pl.BlockSpec((1, h), lambda i: (0, 0)),
            pl.BlockSpec((h, h), lambda i: (0, 0)),
            pl.BlockSpec((1, h), lambda i: (0, 0)),
            pl.BlockSpec((8, h), lambda i: (0, 0)),
        ],
        out_specs=pl.BlockSpec((blk, h), lambda i: (k0 + i, 0)),
        out_shape=jax.ShapeDtypeStruct((rows, h), jnp.float32),
        input_output_aliases={7: 0},
    )(ef_flat, ns_b, m_flat, we, be, ws, bs, prev)


def kernel(node_feat, edge_feat, x_indices, mask_valid, ln_n_g, ln_n_b, W_node,
           b_node, ln_e_g, ln_e_b, W_edge, b_edge, W_skip, b_skip):
    Bn, Nn, Hn = node_feat.shape
    En = edge_feat.shape[1]
    He = edge_feat.shape[2]
    R = Bn * En
    CH = 128
    BLK = 8192

    nf_flat = node_feat.reshape(Bn * Nn, Hn)
    nf_lin = _nf_lin(
        nf_flat,
        ln_n_g.reshape(1, Hn),
        ln_n_b.reshape(1, Hn),
        W_node,
        b_node.reshape(1, He),
        blk=1024,
    )

    x0 = x_indices[0].reshape(En // CH, CH)
    x1 = x_indices[1].reshape(En // CH, CH)
    node_sums = [
        _make_gather_sum_slice(b, En, Nn, He)(nf_lin, x0, x1) for b in range(Bn)
    ]

    we_eff = (ln_e_g[:, None] * W_edge).astype(jnp.bfloat16)
    be_eff = (ln_e_b @ W_edge + b_edge).reshape(1, He)
    ws_bf = W_skip.astype(jnp.bfloat16)
    bs2 = b_skip.reshape(1, He)
    ef_flat = edge_feat.reshape(R, He)
    m_flat = mask_valid.reshape(R, 1)

    out = _edge_mlp_slice_first(ef_flat, node_sums[0], m_flat, we_eff, be_eff,
                                ws_bf, bs2, blk=BLK)
    for b in range(1, Bn):
        out = _edge_mlp_slice_acc(b * (En // BLK), ef_flat, node_sums[b],
                                  m_flat, we_eff, be_eff, ws_bf, bs2, out,
                                  blk=BLK)
    return out.reshape(Bn, En, He)
